# baseline (speedup 1.0000x reference)
import functools

import jax
import jax.numpy as jnp
from jax import lax
from jax.experimental import pallas as pl
from jax.experimental.pallas import tpu as pltpu
from jax.experimental.pallas import tpu_sc as plsc

N = 10000
E = 160000
D = 256
DH = 128
NC = 2
NS = 16
NPAD = 10240

G = 128
CPT = 80
E2 = NS * CPT * G
EPT = E // NS
NFULL = EPT // G
TAIL = EPT - NFULL * G

EPT_C = E // (NC * NS)
NFULL_C = EPT_C // G
TAIL_C = EPT_C - NFULL_C * G

ROWS_T = NPAD // NS
ZR = 128
CW = 128

_mesh = plsc.VectorSubcoreMesh(core_axis_name="c", subcore_axis_name="s")


def _zero_fill2(ref, rows, cols):
    z = jnp.zeros((16,), jnp.float32)
    cpr = cols // 16

    def body(i, _):
        ref[i // cpr, pl.ds((i % cpr) * 16, 16)] = z
        return 0

    lax.fori_loop(0, rows * cpr, body, 0, unroll=4)


@functools.partial(
    pl.kernel,
    out_type=jax.ShapeDtypeStruct((NC, NPAD, DH), jnp.float32),
    mesh=_mesh,
    scratch_types=[
        pltpu.VMEM((G,), jnp.int32),
        pltpu.VMEM((G,), jnp.int32),
        pltpu.VMEM((G, DH), jnp.float32),
        pltpu.VMEM((ZR, DH), jnp.float32),
        pltpu.VMEM_SHARED((NPAD, DH), jnp.float32),
        pltpu.SemaphoreType.DMA,
    ],
)
def _segsum(h2, src, dst, out, src_v, dst_v, rows_v, zbuf, accum, sem):
    c = lax.axis_index("c")
    s = lax.axis_index("s")

    _zero_fill2(zbuf, ZR, DH)
    for k in range(ROWS_T // ZR):
        pltpu.sync_copy(zbuf, accum.at[pl.ds(s * ROWS_T + k * ZR, ZR)])
    plsc.subcore_barrier()

    ebase = s * CPT * G

    def chunk(k, _):
        base = pl.multiple_of(ebase + k * G, 8)
        pltpu.sync_copy(src.at[pl.ds(base, G)], src_v)
        pltpu.async_copy(h2.at[c].at[src_v], rows_v, sem).wait()
        pltpu.sync_copy(dst.at[pl.ds(base, G)], dst_v)
        pltpu.sync_copy(rows_v, accum.at[dst_v], add=True)
        return 0

    lax.fori_loop(0, CPT, chunk, 0)

    plsc.subcore_barrier()
    pltpu.sync_copy(accum.at[pl.ds(s * ROWS_T, ROWS_T)],
                    out.at[c, pl.ds(s * ROWS_T, ROWS_T)])


@functools.partial(
    pl.kernel,
    out_type=jax.ShapeDtypeStruct((NC, NPAD, CW), jnp.float32),
    mesh=_mesh,
    scratch_types=[
        pltpu.VMEM((G,), jnp.int32),
        pltpu.VMEM((G, CW), jnp.float32),
        pltpu.VMEM((TAIL_C,), jnp.int32),
        pltpu.VMEM((TAIL_C, CW), jnp.float32),
        pltpu.VMEM((ZR, CW), jnp.float32),
        pltpu.VMEM_SHARED((NPAD, CW), jnp.float32),
    ],
)
def _counts(dst, out, dst_v, ones_v, tdst_v, tones_v, zbuf, accum):
    c = lax.axis_index("c")
    s = lax.axis_index("s")
    rpt = NPAD // NS

    _zero_fill2(zbuf, ZR, CW)
    for k in range(rpt // ZR):
        pltpu.sync_copy(zbuf, accum.at[pl.ds(s * rpt + k * ZR, ZR)])

    one = jnp.ones((16,), jnp.float32)
    cpr = CW // 16

    def fill_ones(ref, n):
        def body(i, _):
            ref[i // cpr, pl.ds((i % cpr) * 16, 16)] = one
            return 0
        lax.fori_loop(0, n * cpr, body, 0, unroll=4)

    fill_ones(ones_v, G)
    fill_ones(tones_v, TAIL_C)
    plsc.subcore_barrier()

    ebase = c * (E // NC) + s * EPT_C

    def chunk(k, _):
        base = pl.multiple_of(ebase + k * G, 8)
        pltpu.sync_copy(dst.at[pl.ds(base, G)], dst_v)
        pltpu.sync_copy(ones_v, accum.at[dst_v], add=True)
        return 0

    lax.fori_loop(0, NFULL_C, chunk, 0)

    tbase = pl.multiple_of(ebase + NFULL_C * G, 8)
    pltpu.sync_copy(dst.at[pl.ds(tbase, TAIL_C)], tdst_v)
    pltpu.sync_copy(tones_v, accum.at[tdst_v], add=True)

    plsc.subcore_barrier()
    pltpu.sync_copy(accum.at[pl.ds(s * rpt, rpt)],
                    out.at[c, pl.ds(s * rpt, rpt)])


R = 400


def _tc_body(last, sum_ref, cnt_ref, h_ref, wl_ref, wr_ref, b_ref, out_ref):
    cnt = cnt_ref[0, :, 0] + cnt_ref[1, :, 0]
    inv = 1.0 / jnp.maximum(cnt, 1.0)
    m0 = sum_ref[0] * inv[:, None]
    m1 = sum_ref[1] * inv[:, None]
    acc = (
        jnp.dot(m0, wl_ref[0:DH, :], preferred_element_type=jnp.float32)
        + jnp.dot(m1, wl_ref[DH:D, :], preferred_element_type=jnp.float32)
        + jnp.dot(h_ref[0], wr_ref[0:DH, :], preferred_element_type=jnp.float32)
        + jnp.dot(h_ref[1], wr_ref[DH:D, :], preferred_element_type=jnp.float32)
        + b_ref[:][None, :]
    )
    if last:
        out_ref[:, :] = acc
    else:
        acc = jnp.maximum(acc, 0.0)
        out_ref[0] = acc[:, 0:DH]
        out_ref[1] = acc[:, DH:D]


def _tc_layer(summed2, cnt3, h2, wl, wr, b, last):
    in_specs = [
        pl.BlockSpec((NC, R, DH), lambda i: (0, i, 0)),
        pl.BlockSpec((NC, R, 1), lambda i: (0, i, 0)),
        pl.BlockSpec((NC, R, DH), lambda i: (0, i, 0)),
        pl.BlockSpec((D, D), lambda i: (0, 0)),
        pl.BlockSpec((D, D), lambda i: (0, 0)),
        pl.BlockSpec((D,), lambda i: (0,)),
    ]
    if last:
        out_shape = jax.ShapeDtypeStruct((N, D), jnp.float32)
        out_spec = pl.BlockSpec((R, D), lambda i: (i, 0))
    else:
        out_shape = jax.ShapeDtypeStruct((NC, N, DH), jnp.float32)
        out_spec = pl.BlockSpec((NC, R, DH), lambda i: (0, i, 0))
    return pl.pallas_call(
        functools.partial(_tc_body, last),
        grid=(N // R,),
        in_specs=in_specs,
        out_specs=out_spec,
        out_shape=out_shape,
    )(summed2, cnt3, h2, wl, wr, b)


def kernel(x, edge_index, Wl0, Wr0, b0, Wl1, Wr1, b1, Wl2, Wr2, b2,
           Wl3, Wr3, b3):
    src = edge_index[0].astype(jnp.int32)
    dst = edge_index[1].astype(jnp.int32)

    pad = E2 - E
    ar = jnp.arange(pad, dtype=jnp.int32)
    srcp = jnp.concatenate([src, ar % N])
    dstp = jnp.concatenate([dst, N + ar % (NPAD - N)])

    cnt3 = _counts(dst)[:, :N, 0:1]
    h2 = x.reshape(N, NC, DH).transpose(1, 0, 2)

    for wl, wr, b, last in ((Wl0, Wr0, b0, False),
                            (Wl1, Wr1, b1, False),
                            (Wl2, Wr2, b2, False),
                            (Wl3, Wr3, b3, True)):
        summed2 = _segsum(h2, srcp, dstp)
        h2 = _tc_layer(summed2, cnt3, h2, wl, wr, b, last)
    return h2


# speedup vs baseline: 1.6352x; 1.6352x over previous
import functools

import jax
import jax.numpy as jnp
from jax import lax
from jax.experimental import pallas as pl
from jax.experimental.pallas import tpu as pltpu
from jax.experimental.pallas import tpu_sc as plsc

N = 10000
E = 160000
D = 256
DH = 128
NC = 2
NS = 16
NPAD = 10240

G = 128
CPT = 80
HCPT = CPT // 2
E2 = NS * CPT * G
ROWSP = E2 // G
EPT = E // NS
NFULL = EPT // G
TAIL = EPT - NFULL * G

EPT_C = E // (NC * NS)
NFULL_C = EPT_C // G
TAIL_C = EPT_C - NFULL_C * G

ROWS_T = NPAD // NS
ZR = 128
CW = 128

_mesh = plsc.VectorSubcoreMesh(core_axis_name="c", subcore_axis_name="s")


def _zero_fill2(ref, rows, cols):
    z = jnp.zeros((16,), jnp.float32)
    cpr = cols // 16

    def body(i, _):
        ref[i // cpr, pl.ds((i % cpr) * 16, 16)] = z
        return 0

    lax.fori_loop(0, rows * cpr, body, 0, unroll=4)


@functools.partial(
    pl.kernel,
    out_type=jax.ShapeDtypeStruct((NC, NPAD, DH), jnp.float32),
    mesh=_mesh,
    scratch_types=[
        pltpu.VMEM((HCPT, G), jnp.int32),
        pltpu.VMEM((HCPT, G), jnp.int32),
        pltpu.VMEM((G, DH), jnp.float32),
        pltpu.VMEM((G, DH), jnp.float32),
        pltpu.VMEM_SHARED((NPAD, DH), jnp.float32),
        pltpu.SemaphoreType.DMA,
        pltpu.SemaphoreType.DMA,
        pltpu.SemaphoreType.DMA,
        pltpu.SemaphoreType.DMA,
    ],
)
def _segsum(h2, srcp2, dstp2, out, idx_s, idx_d, b0, b1, accum,
            g0, g1, w0, w1):
    c = lax.axis_index("c")
    s = lax.axis_index("s")
    bufs = (b0, b1)
    gsems = (g0, g1)
    wsems = (w0, w1)

    def gather(ci, j):
        return pltpu.async_copy(h2.at[c].at[idx_s.at[ci]], bufs[j], gsems[j])

    def scatter_wait(ci, j):
        pltpu.make_async_copy(bufs[j], accum.at[idx_d.at[ci]],
                              wsems[j]).wait()

    _zero_fill2(b0, G, DH)
    for k in range(ROWS_T // ZR):
        pltpu.sync_copy(b0, accum.at[pl.ds(s * ROWS_T + k * ZR, ZR)])
    plsc.subcore_barrier()

    for half in range(2):
        hb = s * CPT + half * HCPT
        pltpu.sync_copy(srcp2.at[pl.ds(hb, HCPT)], idx_s)
        pltpu.sync_copy(dstp2.at[pl.ds(hb, HCPT)], idx_d)
        gather(0, 0)

        def body(k2, _):
            for j in range(2):
                ci = k2 * 2 + j
                pltpu.make_async_copy(
                    h2.at[c].at[idx_s.at[ci]], bufs[j], gsems[j]).wait()
                pltpu.async_copy(bufs[j], accum.at[idx_d.at[ci]], wsems[j],
                                 add=True)

                @pl.when(ci >= 1)
                def _():
                    scatter_wait(ci - 1, 1 - j)

                @pl.when(ci + 1 < HCPT)
                def _():
                    gather(ci + 1, 1 - j)
            return 0

        lax.fori_loop(0, HCPT // 2, body, 0)
        scatter_wait(HCPT - 1, (HCPT - 1) % 2)

    plsc.subcore_barrier()
    pltpu.sync_copy(accum.at[pl.ds(s * ROWS_T, ROWS_T)],
                    out.at[c, pl.ds(s * ROWS_T, ROWS_T)])


@functools.partial(
    pl.kernel,
    out_type=jax.ShapeDtypeStruct((NC, NPAD, CW), jnp.float32),
    mesh=_mesh,
    scratch_types=[
        pltpu.VMEM((G,), jnp.int32),
        pltpu.VMEM((G, CW), jnp.float32),
        pltpu.VMEM((TAIL_C,), jnp.int32),
        pltpu.VMEM((TAIL_C, CW), jnp.float32),
        pltpu.VMEM((ZR, CW), jnp.float32),
        pltpu.VMEM_SHARED((NPAD, CW), jnp.float32),
    ],
)
def _counts(dst, out, dst_v, ones_v, tdst_v, tones_v, zbuf, accum):
    c = lax.axis_index("c")
    s = lax.axis_index("s")
    rpt = NPAD // NS

    _zero_fill2(zbuf, ZR, CW)
    for k in range(rpt // ZR):
        pltpu.sync_copy(zbuf, accum.at[pl.ds(s * rpt + k * ZR, ZR)])

    one = jnp.ones((16,), jnp.float32)
    cpr = CW // 16

    def fill_ones(ref, n):
        def body(i, _):
            ref[i // cpr, pl.ds((i % cpr) * 16, 16)] = one
            return 0
        lax.fori_loop(0, n * cpr, body, 0, unroll=4)

    fill_ones(ones_v, G)
    fill_ones(tones_v, TAIL_C)
    plsc.subcore_barrier()

    ebase = c * (E // NC) + s * EPT_C

    def chunk(k, _):
        base = pl.multiple_of(ebase + k * G, 8)
        pltpu.sync_copy(dst.at[pl.ds(base, G)], dst_v)
        pltpu.sync_copy(ones_v, accum.at[dst_v], add=True)
        return 0

    lax.fori_loop(0, NFULL_C, chunk, 0)

    tbase = pl.multiple_of(ebase + NFULL_C * G, 8)
    pltpu.sync_copy(dst.at[pl.ds(tbase, TAIL_C)], tdst_v)
    pltpu.sync_copy(tones_v, accum.at[tdst_v], add=True)

    plsc.subcore_barrier()
    pltpu.sync_copy(accum.at[pl.ds(s * rpt, rpt)],
                    out.at[c, pl.ds(s * rpt, rpt)])


R = 400


def _tc_body(last, sum_ref, cnt_ref, h_ref, wl_ref, wr_ref, b_ref, out_ref):
    cnt = cnt_ref[0, :, 0] + cnt_ref[1, :, 0]
    inv = 1.0 / jnp.maximum(cnt, 1.0)
    m0 = sum_ref[0] * inv[:, None]
    m1 = sum_ref[1] * inv[:, None]
    acc = (
        jnp.dot(m0, wl_ref[0:DH, :], preferred_element_type=jnp.float32)
        + jnp.dot(m1, wl_ref[DH:D, :], preferred_element_type=jnp.float32)
        + jnp.dot(h_ref[0], wr_ref[0:DH, :], preferred_element_type=jnp.float32)
        + jnp.dot(h_ref[1], wr_ref[DH:D, :], preferred_element_type=jnp.float32)
        + b_ref[:][None, :]
    )
    if last:
        out_ref[:, :] = acc
    else:
        acc = jnp.maximum(acc, 0.0)
        out_ref[0] = acc[:, 0:DH]
        out_ref[1] = acc[:, DH:D]


def _tc_layer(summed2, cnt3, h2, wl, wr, b, last):
    in_specs = [
        pl.BlockSpec((NC, R, DH), lambda i: (0, i, 0)),
        pl.BlockSpec((NC, R, 1), lambda i: (0, i, 0)),
        pl.BlockSpec((NC, R, DH), lambda i: (0, i, 0)),
        pl.BlockSpec((D, D), lambda i: (0, 0)),
        pl.BlockSpec((D, D), lambda i: (0, 0)),
        pl.BlockSpec((D,), lambda i: (0,)),
    ]
    if last:
        out_shape = jax.ShapeDtypeStruct((N, D), jnp.float32)
        out_spec = pl.BlockSpec((R, D), lambda i: (i, 0))
    else:
        out_shape = jax.ShapeDtypeStruct((NC, N, DH), jnp.float32)
        out_spec = pl.BlockSpec((NC, R, DH), lambda i: (0, i, 0))
    return pl.pallas_call(
        functools.partial(_tc_body, last),
        grid=(N // R,),
        in_specs=in_specs,
        out_specs=out_spec,
        out_shape=out_shape,
    )(summed2, cnt3, h2, wl, wr, b)


def kernel(x, edge_index, Wl0, Wr0, b0, Wl1, Wr1, b1, Wl2, Wr2, b2,
           Wl3, Wr3, b3):
    src = edge_index[0].astype(jnp.int32)
    dst = edge_index[1].astype(jnp.int32)

    pad = E2 - E
    ar = jnp.arange(pad, dtype=jnp.int32)
    srcp = jnp.concatenate([src, ar % N])
    dstp = jnp.concatenate([dst, N + ar % (NPAD - N)])

    cnt3 = _counts(dst)[:, :N, 0:1]
    h2 = x.reshape(N, NC, DH).transpose(1, 0, 2)

    for wl, wr, b, last in ((Wl0, Wr0, b0, False),
                            (Wl1, Wr1, b1, False),
                            (Wl2, Wr2, b2, False),
                            (Wl3, Wr3, b3, True)):
        summed2 = _segsum(h2, srcp.reshape(ROWSP, G), dstp.reshape(ROWSP, G))
        h2 = _tc_layer(summed2, cnt3, h2, wl, wr, b, last)
    return h2


# speedup vs baseline: 1.6649x; 1.0182x over previous
import functools

import jax
import jax.numpy as jnp
from jax import lax
from jax.experimental import pallas as pl
from jax.experimental.pallas import tpu as pltpu
from jax.experimental.pallas import tpu_sc as plsc

N = 10000
E = 160000
D = 256
DH = 128
NC = 2
NS = 16
NPAD = 10240

G = 128
CPT = 80
HCPT = CPT // 2
E2 = NS * CPT * G
ROWSP = E2 // G
CROWS = ROWSP // NC
CPT_C = CROWS // NS

ROWS_T = NPAD // NS
ZR = 128
CW = 128

_mesh = plsc.VectorSubcoreMesh(core_axis_name="c", subcore_axis_name="s")


def _zero_fill2(ref, rows, cols):
    z = jnp.zeros((16,), jnp.float32)
    cpr = cols // 16

    def body(i, _):
        ref[i // cpr, pl.ds((i % cpr) * 16, 16)] = z
        return 0

    lax.fori_loop(0, rows * cpr, body, 0, unroll=4)


@functools.partial(
    pl.kernel,
    out_type=jax.ShapeDtypeStruct((NC, NPAD, DH), jnp.float32),
    mesh=_mesh,
    scratch_types=[
        pltpu.VMEM((HCPT, G), jnp.int32),
        pltpu.VMEM((HCPT, G), jnp.int32),
        pltpu.VMEM((G, DH), jnp.float32),
        pltpu.VMEM((G, DH), jnp.float32),
        pltpu.VMEM_SHARED((NPAD, DH), jnp.float32),
        pltpu.SemaphoreType.DMA,
        pltpu.SemaphoreType.DMA,
        pltpu.SemaphoreType.DMA,
        pltpu.SemaphoreType.DMA,
    ],
)
def _segsum(h2, srcp2, dstp2, out, idx_s, idx_d, b0, b1, accum,
            g0, g1, w0, w1):
    c = lax.axis_index("c")
    s = lax.axis_index("s")
    bufs = (b0, b1)
    gsems = (g0, g1)
    wsems = (w0, w1)

    def gather(ci, j):
        return pltpu.async_copy(h2.at[c].at[idx_s.at[ci]], bufs[j], gsems[j])

    def scatter_wait(ci, j):
        pltpu.make_async_copy(bufs[j], accum.at[idx_d.at[ci]],
                              wsems[j]).wait()

    _zero_fill2(b0, G, DH)
    for k in range(ROWS_T // ZR):
        pltpu.sync_copy(b0, accum.at[pl.ds(s * ROWS_T + k * ZR, ZR)])
    plsc.subcore_barrier()

    for half in range(2):
        hb = s * CPT + half * HCPT
        pltpu.sync_copy(srcp2.at[pl.ds(hb, HCPT)], idx_s)
        pltpu.sync_copy(dstp2.at[pl.ds(hb, HCPT)], idx_d)
        gather(0, 0)

        def body(k2, _):
            for j in range(2):
                ci = k2 * 2 + j
                pltpu.make_async_copy(
                    h2.at[c].at[idx_s.at[ci]], bufs[j], gsems[j]).wait()
                pltpu.async_copy(bufs[j], accum.at[idx_d.at[ci]], wsems[j],
                                 add=True)

                @pl.when(ci >= 1)
                def _():
                    scatter_wait(ci - 1, 1 - j)

                @pl.when(ci + 1 < HCPT)
                def _():
                    gather(ci + 1, 1 - j)
            return 0

        lax.fori_loop(0, HCPT // 2, body, 0)
        scatter_wait(HCPT - 1, (HCPT - 1) % 2)

    plsc.subcore_barrier()
    pltpu.sync_copy(accum.at[pl.ds(s * ROWS_T, ROWS_T)],
                    out.at[c, pl.ds(s * ROWS_T, ROWS_T)])


@functools.partial(
    pl.kernel,
    out_type=jax.ShapeDtypeStruct((NC, NPAD, CW), jnp.float32),
    mesh=_mesh,
    scratch_types=[
        pltpu.VMEM((CPT_C, G), jnp.int32),
        pltpu.VMEM((G, CW), jnp.float32),
        pltpu.VMEM((ZR, CW), jnp.float32),
        pltpu.VMEM_SHARED((NPAD, CW), jnp.float32),
        pltpu.SemaphoreType.DMA,
        pltpu.SemaphoreType.DMA,
    ],
)
def _counts(dstp2, out, idx_d, ones_v, zbuf, accum, w0, w1):
    c = lax.axis_index("c")
    s = lax.axis_index("s")
    wsems = (w0, w1)

    base = c * CROWS + s * CPT_C
    pltpu.sync_copy(dstp2.at[pl.ds(base, CPT_C)], idx_d)

    _zero_fill2(zbuf, ZR, CW)
    for k in range(ROWS_T // ZR):
        pltpu.sync_copy(zbuf, accum.at[pl.ds(s * ROWS_T + k * ZR, ZR)])

    one = jnp.ones((16,), jnp.float32)
    cpr = CW // 16

    def fill_ones(i, _):
        ones_v[i // cpr, pl.ds((i % cpr) * 16, 16)] = one
        return 0

    lax.fori_loop(0, G * cpr, fill_ones, 0, unroll=4)
    plsc.subcore_barrier()

    def body(k2, _):
        for j in range(2):
            ci = k2 * 2 + j

            @pl.when(ci >= 2)
            def _():
                pltpu.make_async_copy(ones_v, accum.at[idx_d.at[ci - 2]],
                                      wsems[j]).wait()

            pltpu.async_copy(ones_v, accum.at[idx_d.at[ci]], wsems[j],
                             add=True)
        return 0

    lax.fori_loop(0, CPT_C // 2, body, 0)
    for j in range(2):
        pltpu.make_async_copy(ones_v, accum.at[idx_d.at[CPT_C - 2 + j]],
                              wsems[j]).wait()

    plsc.subcore_barrier()
    pltpu.sync_copy(accum.at[pl.ds(s * ROWS_T, ROWS_T)],
                    out.at[c, pl.ds(s * ROWS_T, ROWS_T)])


R = 400


def _tc_body(last, sum_ref, cnt_ref, h_ref, wl_ref, wr_ref, b_ref, out_ref):
    cnt = cnt_ref[0, :, 0] + cnt_ref[1, :, 0]
    inv = 1.0 / jnp.maximum(cnt, 1.0)
    m0 = sum_ref[0] * inv[:, None]
    m1 = sum_ref[1] * inv[:, None]
    acc = (
        jnp.dot(m0, wl_ref[0:DH, :], preferred_element_type=jnp.float32)
        + jnp.dot(m1, wl_ref[DH:D, :], preferred_element_type=jnp.float32)
        + jnp.dot(h_ref[0], wr_ref[0:DH, :], preferred_element_type=jnp.float32)
        + jnp.dot(h_ref[1], wr_ref[DH:D, :], preferred_element_type=jnp.float32)
        + b_ref[:][None, :]
    )
    if last:
        out_ref[:, :] = acc
    else:
        acc = jnp.maximum(acc, 0.0)
        out_ref[0] = acc[:, 0:DH]
        out_ref[1] = acc[:, DH:D]


def _tc_layer(summed2, cnt3, h2, wl, wr, b, last):
    in_specs = [
        pl.BlockSpec((NC, R, DH), lambda i: (0, i, 0)),
        pl.BlockSpec((NC, R, 1), lambda i: (0, i, 0)),
        pl.BlockSpec((NC, R, DH), lambda i: (0, i, 0)),
        pl.BlockSpec((D, D), lambda i: (0, 0)),
        pl.BlockSpec((D, D), lambda i: (0, 0)),
        pl.BlockSpec((D,), lambda i: (0,)),
    ]
    if last:
        out_shape = jax.ShapeDtypeStruct((N, D), jnp.float32)
        out_spec = pl.BlockSpec((R, D), lambda i: (i, 0))
    else:
        out_shape = jax.ShapeDtypeStruct((NC, N, DH), jnp.float32)
        out_spec = pl.BlockSpec((NC, R, DH), lambda i: (0, i, 0))
    return pl.pallas_call(
        functools.partial(_tc_body, last),
        grid=(N // R,),
        in_specs=in_specs,
        out_specs=out_spec,
        out_shape=out_shape,
    )(summed2, cnt3, h2, wl, wr, b)


def kernel(x, edge_index, Wl0, Wr0, b0, Wl1, Wr1, b1, Wl2, Wr2, b2,
           Wl3, Wr3, b3):
    src = edge_index[0].astype(jnp.int32)
    dst = edge_index[1].astype(jnp.int32)

    pad = E2 - E
    ar = jnp.arange(pad, dtype=jnp.int32)
    srcp = jnp.concatenate([src, ar % N])
    dstp = jnp.concatenate([dst, N + ar % (NPAD - N)])

    cnt3 = _counts(dstp.reshape(ROWSP, G))[:, :N, 0:1]
    h2 = x.reshape(N, NC, DH).transpose(1, 0, 2)

    for wl, wr, b, last in ((Wl0, Wr0, b0, False),
                            (Wl1, Wr1, b1, False),
                            (Wl2, Wr2, b2, False),
                            (Wl3, Wr3, b3, True)):
        summed2 = _segsum(h2, srcp.reshape(ROWSP, G), dstp.reshape(ROWSP, G))
        h2 = _tc_layer(summed2, cnt3, h2, wl, wr, b, last)
    return h2
